# Initial kernel scaffold; baseline (speedup 1.0000x reference)
import functools

import jax
import jax.numpy as jnp
from jax import lax
from jax.experimental import pallas as pl
from jax.experimental.pallas import tpu as pltpu
from jax.experimental.pallas import tpu_sc as plsc

_NC = 2
_NS = 16
_NW = _NC * _NS

_CHUNK = 128
_N_PAD = 10240
_ROWS_PER_TILE = _N_PAD // _NS


def _linear_tc(h, W, b):
    n, d_in = h.shape
    d_out = W.shape[0]
    blk = 2000
    assert n % blk == 0

    def body(h_ref, w_ref, b_ref, o_ref):
        o_ref[...] = lax.dot_general(
            h_ref[...], w_ref[...],
            (((1,), (1,)), ((), ())),
            preferred_element_type=jnp.float32,
            precision=lax.Precision.HIGHEST,
        ) + b_ref[...]

    return pl.pallas_call(
        body,
        grid=(n // blk,),
        in_specs=[
            pl.BlockSpec((blk, d_in), lambda i: (i, 0)),
            pl.BlockSpec((d_out, d_in), lambda i: (0, 0)),
            pl.BlockSpec((1, d_out), lambda i: (0, 0)),
        ],
        out_specs=pl.BlockSpec((blk, d_out), lambda i: (i, 0)),
        out_shape=jax.ShapeDtypeStruct((n, d_out), jnp.float32),
    )(h, W, b.reshape(1, d_out))


def _make_sc_agg(cpt, d):
    mesh = plsc.VectorSubcoreMesh(core_axis_name="c", subcore_axis_name="s")

    @functools.partial(
        pl.kernel,
        out_type=[
            jax.ShapeDtypeStruct((_NC * _N_PAD, d), jnp.float32),
            jax.ShapeDtypeStruct((_NC * _N_PAD, 16), jnp.float32),
        ],
        mesh=mesh,
        scratch_types=[
            pltpu.VMEM((cpt, _CHUNK), jnp.int32),
            pltpu.VMEM((cpt, _CHUNK), jnp.int32),
            pltpu.VMEM((_CHUNK, d), jnp.float32),
            pltpu.VMEM((_CHUNK, 16), jnp.float32),
            pltpu.VMEM((_ROWS_PER_TILE, 16), jnp.float32),
            pltpu.VMEM_SHARED((_N_PAD, d), jnp.float32),
            pltpu.VMEM_SHARED((_N_PAD, 16), jnp.float32),
        ],
    )
    def sc_agg(hlin_hbm, src_hbm, dst_hbm, acc_out, deg_out,
               src_v, dst_v, rows_v, ones_v, z16_v, acc_sh, deg_sh):
        c = lax.axis_index("c")
        s = lax.axis_index("s")
        wid = s * _NC + c

        @pl.loop(0, _CHUNK)
        def _(i):
            ones_v[i, pl.ds(0, 16)] = jnp.ones((16,), jnp.float32)

            @pl.loop(0, d // 16)
            def _(j):
                rows_v[i, pl.ds(j * 16, 16)] = jnp.zeros((16,), jnp.float32)

        @pl.loop(0, _ROWS_PER_TILE)
        def _(i):
            z16_v[i, pl.ds(0, 16)] = jnp.zeros((16,), jnp.float32)

        base = s * _ROWS_PER_TILE

        @pl.loop(0, _ROWS_PER_TILE // _CHUNK)
        def _(k):
            pltpu.sync_copy(rows_v, acc_sh.at[pl.ds(base + k * _CHUNK, _CHUNK)])

        pltpu.sync_copy(z16_v, deg_sh.at[pl.ds(base, _ROWS_PER_TILE)])
        plsc.subcore_barrier()

        pltpu.sync_copy(src_hbm.at[pl.ds(wid * cpt, cpt)], src_v)
        pltpu.sync_copy(dst_hbm.at[pl.ds(wid * cpt, cpt)], dst_v)

        @pl.loop(0, cpt)
        def _(j):
            pltpu.sync_copy(hlin_hbm.at[src_v.at[j]], rows_v)
            pltpu.sync_copy(rows_v, acc_sh.at[dst_v.at[j]], add=True)
            pltpu.sync_copy(ones_v, deg_sh.at[dst_v.at[j]], add=True)

        plsc.subcore_barrier()

        out_base = c * _N_PAD + base
        pltpu.sync_copy(acc_sh.at[pl.ds(base, _ROWS_PER_TILE)],
                        acc_out.at[pl.ds(out_base, _ROWS_PER_TILE)])
        pltpu.sync_copy(deg_sh.at[pl.ds(base, _ROWS_PER_TILE)],
                        deg_out.at[pl.ds(out_base, _ROWS_PER_TILE)])

    return sc_agg


def _finalize_tc(acc, deg, n, d):
    blk = 2000
    assert n % blk == 0
    acc3 = acc.reshape(_NC, _N_PAD, d)
    deg3 = deg.reshape(_NC, _N_PAD, 16)

    def body(a_ref, g_ref, o_ref):
        a = a_ref[0] + a_ref[1]
        dsum = g_ref[0, :, 0:1] + g_ref[1, :, 0:1]
        o_ref[...] = a / jnp.maximum(dsum, 1.0)

    return pl.pallas_call(
        body,
        grid=(n // blk,),
        in_specs=[
            pl.BlockSpec((_NC, blk, d), lambda i: (0, i, 0)),
            pl.BlockSpec((_NC, blk, 16), lambda i: (0, i, 0)),
        ],
        out_specs=pl.BlockSpec((blk, d), lambda i: (i, 0)),
        out_shape=jax.ShapeDtypeStruct((n, d), jnp.float32),
    )(acc3, deg3)


def kernel(h, edge_index, W, b):
    n, d_in = h.shape
    d = W.shape[0]
    e = edge_index.shape[1]

    h_lin = _linear_tc(h, W, b)

    chunks = -(-e // _CHUNK)
    cpt = -(-chunks // _NW)
    e_pad = cpt * _NW * _CHUNK
    src = edge_index[0].astype(jnp.int32)
    dst = edge_index[1].astype(jnp.int32)
    pad = e_pad - e
    src_p = jnp.concatenate([src, jnp.zeros((pad,), jnp.int32)])
    dst_p = jnp.concatenate([dst, jnp.full((pad,), _N_PAD - 8, jnp.int32)])
    src2 = src_p.reshape(cpt * _NW, _CHUNK)
    dst2 = dst_p.reshape(cpt * _NW, _CHUNK)

    acc, deg = _make_sc_agg(cpt, d)(h_lin, src2, dst2)
    return _finalize_tc(acc, deg, n, d)


# speedup vs baseline: 3.8871x; 3.8871x over previous
import functools

import jax
import jax.numpy as jnp
from jax import lax
from jax.experimental import pallas as pl
from jax.experimental.pallas import tpu as pltpu
from jax.experimental.pallas import tpu_sc as plsc

_NC = 2
_NS = 16
_NW = _NC * _NS

_CHUNK = 128
_IGRP = 16
_N_PAD = 10240
_ROWS_PER_TILE = _N_PAD // _NS


def _linear_tc(h, W, b):
    n, d_in = h.shape
    d_out = W.shape[0]
    blk = 2000
    assert n % blk == 0

    def body(h_ref, w_ref, b_ref, o_ref):
        o_ref[...] = lax.dot_general(
            h_ref[...], w_ref[...],
            (((1,), (1,)), ((), ())),
            preferred_element_type=jnp.float32,
            precision=lax.Precision.HIGHEST,
        ) + b_ref[...]

    return pl.pallas_call(
        body,
        grid=(n // blk,),
        in_specs=[
            pl.BlockSpec((blk, d_in), lambda i: (i, 0)),
            pl.BlockSpec((d_out, d_in), lambda i: (0, 0)),
            pl.BlockSpec((1, d_out), lambda i: (0, 0)),
        ],
        out_specs=pl.BlockSpec((blk, d_out), lambda i: (i, 0)),
        out_shape=jax.ShapeDtypeStruct((n, d_out), jnp.float32),
    )(h, W, b.reshape(1, d_out))


def _make_sc_agg(cpt, d):
    mesh = plsc.VectorSubcoreMesh(core_axis_name="c", subcore_axis_name="s")

    @functools.partial(
        pl.kernel,
        out_type=[
            jax.ShapeDtypeStruct((_NC * _N_PAD, d), jnp.float32),
            jax.ShapeDtypeStruct((_NC * _N_PAD, 16), jnp.float32),
        ],
        mesh=mesh,
        compiler_params=pltpu.CompilerParams(use_tc_tiling_on_sc=False),
        scratch_types=[
            pltpu.VMEM((_IGRP, _CHUNK), jnp.int32),
            pltpu.VMEM((_IGRP, _CHUNK), jnp.int32),
            pltpu.VMEM((_CHUNK, d), jnp.float32),
            pltpu.VMEM((_CHUNK, 16), jnp.float32),
            pltpu.VMEM((_CHUNK, 16), jnp.float32),
            pltpu.VMEM_SHARED((_N_PAD, d), jnp.float32),
            pltpu.VMEM_SHARED((_N_PAD, 16), jnp.float32),
        ],
    )
    def sc_agg(hlin_hbm, src_hbm, dst_hbm, acc_out, deg_out,
               src_v, dst_v, rows_v, ones_v, z16_v, acc_sh, deg_sh):
        c = lax.axis_index("c")
        s = lax.axis_index("s")
        wid = s * _NC + c

        @pl.loop(0, _CHUNK)
        def _(i):
            ones_v[i, pl.ds(0, 16)] = jnp.ones((16,), jnp.float32)
            z16_v[i, pl.ds(0, 16)] = jnp.zeros((16,), jnp.float32)

            @pl.loop(0, d // 16)
            def _(j):
                rows_v[i, pl.ds(j * 16, 16)] = jnp.zeros((16,), jnp.float32)

        base = s * _ROWS_PER_TILE

        @pl.loop(0, _ROWS_PER_TILE // _CHUNK)
        def _(k):
            pltpu.sync_copy(rows_v, acc_sh.at[pl.ds(base + k * _CHUNK, _CHUNK)])
            pltpu.sync_copy(z16_v, deg_sh.at[pl.ds(base + k * _CHUNK, _CHUNK)])

        plsc.subcore_barrier()

        @pl.loop(0, cpt // _IGRP)
        def _(g):
            gbase = wid * cpt + g * _IGRP
            pltpu.sync_copy(src_hbm.at[pl.ds(gbase, _IGRP)], src_v)
            pltpu.sync_copy(dst_hbm.at[pl.ds(gbase, _IGRP)], dst_v)

            @pl.loop(0, _IGRP)
            def _(j):
                pltpu.sync_copy(hlin_hbm.at[src_v.at[j]], rows_v)
                pltpu.sync_copy(rows_v, acc_sh.at[dst_v.at[j]], add=True)
                pltpu.sync_copy(ones_v, deg_sh.at[dst_v.at[j]], add=True)

        plsc.subcore_barrier()

        out_base = c * _N_PAD + base
        pltpu.sync_copy(acc_sh.at[pl.ds(base, _ROWS_PER_TILE)],
                        acc_out.at[pl.ds(out_base, _ROWS_PER_TILE)])
        pltpu.sync_copy(deg_sh.at[pl.ds(base, _ROWS_PER_TILE)],
                        deg_out.at[pl.ds(out_base, _ROWS_PER_TILE)])

    return sc_agg


def _finalize_tc(acc, deg, n, d):
    blk = 2000
    assert n % blk == 0
    acc3 = acc.reshape(_NC, _N_PAD, d)
    deg3 = deg.reshape(_NC, _N_PAD, 16)

    def body(a_ref, g_ref, o_ref):
        a = a_ref[0] + a_ref[1]
        dsum = g_ref[0, :, 0:1] + g_ref[1, :, 0:1]
        o_ref[...] = a / jnp.maximum(dsum, 1.0)

    return pl.pallas_call(
        body,
        grid=(n // blk,),
        in_specs=[
            pl.BlockSpec((_NC, blk, d), lambda i: (0, i, 0)),
            pl.BlockSpec((_NC, blk, 16), lambda i: (0, i, 0)),
        ],
        out_specs=pl.BlockSpec((blk, d), lambda i: (i, 0)),
        out_shape=jax.ShapeDtypeStruct((n, d), jnp.float32),
    )(acc3, deg3)


def kernel(h, edge_index, W, b):
    n, d_in = h.shape
    d = W.shape[0]
    e = edge_index.shape[1]

    h_lin = _linear_tc(h, W, b)

    chunks = -(-e // _CHUNK)
    cpt = -(-chunks // _NW)
    cpt = -(-cpt // 8) * 8
    e_pad = cpt * _NW * _CHUNK
    src = edge_index[0].astype(jnp.int32)
    dst = edge_index[1].astype(jnp.int32)
    pad = e_pad - e
    src_p = jnp.concatenate([src, jnp.zeros((pad,), jnp.int32)])
    dst_p = jnp.concatenate([dst, jnp.full((pad,), _N_PAD - 8, jnp.int32)])
    src2 = src_p.reshape(cpt * _NW, _CHUNK)
    dst2 = dst_p.reshape(cpt * _NW, _CHUNK)

    acc, deg = _make_sc_agg(cpt, d)(h_lin, src2, dst2)
    return _finalize_tc(acc, deg, n, d)


# speedup vs baseline: 4.0119x; 1.0321x over previous
import functools

import jax
import jax.numpy as jnp
from jax import lax
from jax.experimental import pallas as pl
from jax.experimental.pallas import tpu as pltpu
from jax.experimental.pallas import tpu_sc as plsc

_NC = 2
_NS = 16
_NW = _NC * _NS

_CHUNK = 128
_IGRP = 16
_N_PAD = 10240
_ROWS_PER_TILE = _N_PAD // _NS


def _linear_tc(h, W, b):
    n, d_in = h.shape
    d_out = W.shape[0]
    blk = 2000
    assert n % blk == 0

    def body(h_ref, w_ref, b_ref, o_ref):
        o_ref[...] = lax.dot_general(
            h_ref[...], w_ref[...],
            (((1,), (1,)), ((), ())),
            preferred_element_type=jnp.float32,
            precision=lax.Precision.HIGHEST,
        ) + b_ref[...]

    return pl.pallas_call(
        body,
        grid=(n // blk,),
        in_specs=[
            pl.BlockSpec((blk, d_in), lambda i: (i, 0)),
            pl.BlockSpec((d_out, d_in), lambda i: (0, 0)),
            pl.BlockSpec((1, d_out), lambda i: (0, 0)),
        ],
        out_specs=pl.BlockSpec((blk, d_out), lambda i: (i, 0)),
        out_shape=jax.ShapeDtypeStruct((n, d_out), jnp.float32),
    )(h, W, b.reshape(1, d_out))


def _make_sc_agg(cpt, d):
    mesh = plsc.VectorSubcoreMesh(core_axis_name="c", subcore_axis_name="s")

    @functools.partial(
        pl.kernel,
        out_type=[
            jax.ShapeDtypeStruct((_NC * _N_PAD, d), jnp.float32),
            jax.ShapeDtypeStruct((_NC * _N_PAD, 16), jnp.float32),
        ],
        mesh=mesh,
        compiler_params=pltpu.CompilerParams(use_tc_tiling_on_sc=False),
        scratch_types=[
            pltpu.VMEM((1, _CHUNK), jnp.int32),
            pltpu.VMEM((1, _CHUNK), jnp.int32),
            pltpu.VMEM((1, _CHUNK), jnp.int32),
            pltpu.VMEM((1, _CHUNK), jnp.int32),
            pltpu.VMEM((_CHUNK, d), jnp.float32),
            pltpu.VMEM((_CHUNK, d), jnp.float32),
            pltpu.VMEM((_CHUNK, 16), jnp.float32),
            pltpu.VMEM((_CHUNK, 16), jnp.float32),
            pltpu.VMEM_SHARED((_N_PAD, d), jnp.float32),
            pltpu.VMEM_SHARED((_N_PAD, 16), jnp.float32),
            pltpu.SemaphoreType.DMA,
            pltpu.SemaphoreType.DMA,
            pltpu.SemaphoreType.DMA,
            pltpu.SemaphoreType.DMA,
            pltpu.SemaphoreType.DMA,
            pltpu.SemaphoreType.DMA,
        ],
    )
    def sc_agg(hlin_hbm, src_hbm, dst_hbm, acc_out, deg_out,
               sa0, sa1, da0, da1, rows0, rows1, ones_v, z16_v,
               acc_sh, deg_sh, g0, g1, si0, si1, di0, di1):
        c = lax.axis_index("c")
        s = lax.axis_index("s")
        wid = s * _NC + c
        t0 = wid * cpt

        @pl.loop(0, _CHUNK)
        def _(i):
            ones_v[i, pl.ds(0, 16)] = jnp.ones((16,), jnp.float32)
            z16_v[i, pl.ds(0, 16)] = jnp.zeros((16,), jnp.float32)

            @pl.loop(0, d // 16)
            def _(j):
                rows0[i, pl.ds(j * 16, 16)] = jnp.zeros((16,), jnp.float32)

        base = s * _ROWS_PER_TILE

        @pl.loop(0, _ROWS_PER_TILE // _CHUNK)
        def _(k):
            pltpu.sync_copy(rows0, acc_sh.at[pl.ds(base + k * _CHUNK, _CHUNK)])
            pltpu.sync_copy(z16_v, deg_sh.at[pl.ds(base + k * _CHUNK, _CHUNK)])

        plsc.subcore_barrier()

        def idx_start(j, sa, da, si, di):
            pltpu.async_copy(src_hbm.at[pl.ds(t0 + j, 1)], sa, si)
            pltpu.async_copy(dst_hbm.at[pl.ds(t0 + j, 1)], da, di)

        def idx_wait(j, sa, da, si, di):
            pltpu.make_async_copy(src_hbm.at[pl.ds(t0 + j, 1)], sa, si).wait()
            pltpu.make_async_copy(dst_hbm.at[pl.ds(t0 + j, 1)], da, di).wait()

        def gather_start(sa, rows, g):
            pltpu.async_copy(hlin_hbm.at[sa.at[0]], rows, g)

        def gather_wait(sa, rows, g):
            pltpu.make_async_copy(hlin_hbm.at[sa.at[0]], rows, g).wait()

        def scatter(rows, da):
            pltpu.sync_copy(rows, acc_sh.at[da.at[0]], add=True)
            pltpu.sync_copy(ones_v, deg_sh.at[da.at[0]], add=True)

        idx_start(0, sa0, da0, si0, di0)
        idx_start(1, sa1, da1, si1, di1)
        idx_wait(0, sa0, da0, si0, di0)
        gather_start(sa0, rows0, g0)

        @pl.loop(0, cpt // 2)
        def _(i):
            j0 = 2 * i
            j1 = j0 + 1
            gather_wait(sa0, rows0, g0)
            scatter(rows0, da0)

            @pl.when(j0 + 2 < cpt)
            def _():
                idx_start(j0 + 2, sa0, da0, si0, di0)

            idx_wait(j1, sa1, da1, si1, di1)
            gather_start(sa1, rows1, g1)
            gather_wait(sa1, rows1, g1)
            scatter(rows1, da1)

            @pl.when(j1 + 2 < cpt)
            def _():
                idx_start(j1 + 2, sa1, da1, si1, di1)

            @pl.when(j1 + 1 < cpt)
            def _():
                idx_wait(j1 + 1, sa0, da0, si0, di0)
                gather_start(sa0, rows0, g0)

        plsc.subcore_barrier()

        out_base = c * _N_PAD + base
        pltpu.sync_copy(acc_sh.at[pl.ds(base, _ROWS_PER_TILE)],
                        acc_out.at[pl.ds(out_base, _ROWS_PER_TILE)])
        pltpu.sync_copy(deg_sh.at[pl.ds(base, _ROWS_PER_TILE)],
                        deg_out.at[pl.ds(out_base, _ROWS_PER_TILE)])

    return sc_agg


def _finalize_tc(acc, deg, n, d):
    blk = 2000
    assert n % blk == 0
    acc3 = acc.reshape(_NC, _N_PAD, d)
    deg3 = deg.reshape(_NC, _N_PAD, 16)

    def body(a_ref, g_ref, o_ref):
        a = a_ref[0] + a_ref[1]
        dsum = g_ref[0, :, 0:1] + g_ref[1, :, 0:1]
        o_ref[...] = a / jnp.maximum(dsum, 1.0)

    return pl.pallas_call(
        body,
        grid=(n // blk,),
        in_specs=[
            pl.BlockSpec((_NC, blk, d), lambda i: (0, i, 0)),
            pl.BlockSpec((_NC, blk, 16), lambda i: (0, i, 0)),
        ],
        out_specs=pl.BlockSpec((blk, d), lambda i: (i, 0)),
        out_shape=jax.ShapeDtypeStruct((n, d), jnp.float32),
    )(acc3, deg3)


def kernel(h, edge_index, W, b):
    n, d_in = h.shape
    d = W.shape[0]
    e = edge_index.shape[1]

    h_lin = _linear_tc(h, W, b)

    chunks = -(-e // _CHUNK)
    cpt = -(-chunks // _NW)
    cpt = -(-cpt // 8) * 8
    e_pad = cpt * _NW * _CHUNK
    src = edge_index[0].astype(jnp.int32)
    dst = edge_index[1].astype(jnp.int32)
    pad = e_pad - e
    src_p = jnp.concatenate([src, jnp.zeros((pad,), jnp.int32)])
    dst_p = jnp.concatenate([dst, jnp.full((pad,), _N_PAD - 8, jnp.int32)])
    src2 = src_p.reshape(cpt * _NW, _CHUNK)
    dst2 = dst_p.reshape(cpt * _NW, _CHUNK)

    acc, deg = _make_sc_agg(cpt, d)(h_lin, src2, dst2)
    return _finalize_tc(acc, deg, n, d)


# speedup vs baseline: 4.6036x; 1.1475x over previous
import functools

import jax
import jax.numpy as jnp
from jax import lax
from jax.experimental import pallas as pl
from jax.experimental.pallas import tpu as pltpu
from jax.experimental.pallas import tpu_sc as plsc

_NC = 2
_NS = 16
_NW = _NC * _NS

_CHUNK = 128
_IGRP = 16
_N_PAD = 10240
_ROWS_PER_TILE = _N_PAD // _NS


def _linear_tc(h, W, b):
    n, d_in = h.shape
    d_out = W.shape[0]
    blk = 2000
    assert n % blk == 0

    def body(h_ref, w_ref, b_ref, o_ref):
        o_ref[...] = lax.dot_general(
            h_ref[...], w_ref[...],
            (((1,), (1,)), ((), ())),
            preferred_element_type=jnp.float32,
            precision=lax.Precision.HIGHEST,
        ) + b_ref[...]

    return pl.pallas_call(
        body,
        grid=(n // blk,),
        in_specs=[
            pl.BlockSpec((blk, d_in), lambda i: (i, 0)),
            pl.BlockSpec((d_out, d_in), lambda i: (0, 0)),
            pl.BlockSpec((1, d_out), lambda i: (0, 0)),
        ],
        out_specs=pl.BlockSpec((blk, d_out), lambda i: (i, 0)),
        out_shape=jax.ShapeDtypeStruct((n, d_out), jnp.float32),
    )(h, W, b.reshape(1, d_out))


def _make_sc_agg(cpt, d):
    mesh = plsc.VectorSubcoreMesh(core_axis_name="c", subcore_axis_name="s")

    idx_bufs = [pltpu.VMEM((1, _CHUNK), jnp.int32)] * 8
    idx_sems = [pltpu.SemaphoreType.DMA] * 8

    @functools.partial(
        pl.kernel,
        out_type=[
            jax.ShapeDtypeStruct((_NC * _N_PAD, d), jnp.float32),
            jax.ShapeDtypeStruct((_NC * _N_PAD, 16), jnp.float32),
        ],
        mesh=mesh,
        compiler_params=pltpu.CompilerParams(use_tc_tiling_on_sc=False),
        scratch_types=[
            *idx_bufs,
            pltpu.VMEM((_CHUNK, d), jnp.float32),
            pltpu.VMEM((_CHUNK, d), jnp.float32),
            pltpu.VMEM((_CHUNK, 16), jnp.float32),
            pltpu.VMEM((_CHUNK, 16), jnp.float32),
            pltpu.VMEM_SHARED((_N_PAD, d), jnp.float32),
            pltpu.VMEM_SHARED((_N_PAD, 16), jnp.float32),
            pltpu.SemaphoreType.DMA,
            pltpu.SemaphoreType.DMA,
            *idx_sems,
        ],
    )
    def sc_agg(hlin_hbm, src_hbm, dst_hbm, acc_out, deg_out,
               sa0, sa1, sa2, sa3, da0, da1, da2, da3,
               rows0, rows1, ones_v, z16_v,
               acc_sh, deg_sh, g0, g1,
               si0, si1, si2, si3, di0, di1, di2, di3):
        c = lax.axis_index("c")
        s = lax.axis_index("s")
        wid = s * _NC + c
        t0 = wid * cpt

        @pl.loop(0, _CHUNK)
        def _(i):
            ones_v[i, pl.ds(0, 16)] = jnp.ones((16,), jnp.float32)
            z16_v[i, pl.ds(0, 16)] = jnp.zeros((16,), jnp.float32)

            @pl.loop(0, d // 16)
            def _(j):
                rows0[i, pl.ds(j * 16, 16)] = jnp.zeros((16,), jnp.float32)

        base = s * _ROWS_PER_TILE

        @pl.loop(0, _ROWS_PER_TILE // _CHUNK)
        def _(k):
            pltpu.sync_copy(rows0, acc_sh.at[pl.ds(base + k * _CHUNK, _CHUNK)])
            pltpu.sync_copy(z16_v, deg_sh.at[pl.ds(base + k * _CHUNK, _CHUNK)])

        plsc.subcore_barrier()

        sas = [sa0, sa1, sa2, sa3]
        das = [da0, da1, da2, da3]
        sis = [si0, si1, si2, si3]
        dis = [di0, di1, di2, di3]
        rows = [rows0, rows1]
        gsems = [g0, g1]

        def idx_start(j, k):
            pltpu.async_copy(src_hbm.at[pl.ds(t0 + j, 1)], sas[k], sis[k])
            pltpu.async_copy(dst_hbm.at[pl.ds(t0 + j, 1)], das[k], dis[k])

        def idx_wait(j, k):
            pltpu.make_async_copy(
                src_hbm.at[pl.ds(t0 + j, 1)], sas[k], sis[k]).wait()
            pltpu.make_async_copy(
                dst_hbm.at[pl.ds(t0 + j, 1)], das[k], dis[k]).wait()

        def gather_start(k4, k2):
            pltpu.async_copy(hlin_hbm.at[sas[k4].at[0]], rows[k2], gsems[k2])

        def gather_wait(k4, k2):
            pltpu.make_async_copy(
                hlin_hbm.at[sas[k4].at[0]], rows[k2], gsems[k2]).wait()

        for k in range(4):
            idx_start(k, k)
        idx_wait(0, 0)
        gather_start(0, 0)

        @pl.loop(0, cpt // 4)
        def _(i):
            j0 = 4 * i
            for k in range(4):
                j = j0 + k
                kn = (k + 1) % 4

                @pl.when(j + 1 < cpt)
                def _():
                    idx_wait(j + 1, kn)
                    gather_start(kn, (k + 1) % 2)

                gather_wait(k, k % 2)
                pltpu.sync_copy(rows[k % 2], acc_sh.at[das[k].at[0]],
                                add=True)
                pltpu.sync_copy(ones_v, deg_sh.at[das[k].at[0]], add=True)

                @pl.when(j + 4 < cpt)
                def _():
                    idx_start(j + 4, k)

        plsc.subcore_barrier()

        out_base = c * _N_PAD + base
        pltpu.sync_copy(acc_sh.at[pl.ds(base, _ROWS_PER_TILE)],
                        acc_out.at[pl.ds(out_base, _ROWS_PER_TILE)])
        pltpu.sync_copy(deg_sh.at[pl.ds(base, _ROWS_PER_TILE)],
                        deg_out.at[pl.ds(out_base, _ROWS_PER_TILE)])

    return sc_agg


def _finalize_tc(acc, deg, n, d):
    blk = 2000
    assert n % blk == 0
    acc3 = acc.reshape(_NC, _N_PAD, d)
    deg3 = deg.reshape(_NC, _N_PAD, 16)

    def body(a_ref, g_ref, o_ref):
        a = a_ref[0] + a_ref[1]
        dsum = g_ref[0, :, 0:1] + g_ref[1, :, 0:1]
        o_ref[...] = a / jnp.maximum(dsum, 1.0)

    return pl.pallas_call(
        body,
        grid=(n // blk,),
        in_specs=[
            pl.BlockSpec((_NC, blk, d), lambda i: (0, i, 0)),
            pl.BlockSpec((_NC, blk, 16), lambda i: (0, i, 0)),
        ],
        out_specs=pl.BlockSpec((blk, d), lambda i: (i, 0)),
        out_shape=jax.ShapeDtypeStruct((n, d), jnp.float32),
    )(acc3, deg3)


def kernel(h, edge_index, W, b):
    n, d_in = h.shape
    d = W.shape[0]
    e = edge_index.shape[1]

    h_lin = _linear_tc(h, W, b)

    chunks = -(-e // _CHUNK)
    cpt = -(-chunks // _NW)
    cpt = -(-cpt // 8) * 8
    e_pad = cpt * _NW * _CHUNK
    src = edge_index[0].astype(jnp.int32)
    dst = edge_index[1].astype(jnp.int32)
    pad = e_pad - e
    src_p = jnp.concatenate([src, jnp.zeros((pad,), jnp.int32)])
    dst_p = jnp.concatenate([dst, jnp.full((pad,), _N_PAD - 8, jnp.int32)])
    src2 = src_p.reshape(cpt * _NW, _CHUNK)
    dst2 = dst_p.reshape(cpt * _NW, _CHUNK)

    acc, deg = _make_sc_agg(cpt, d)(h_lin, src2, dst2)
    return _finalize_tc(acc, deg, n, d)
